# Initial kernel scaffold; baseline (speedup 1.0000x reference)
import functools

import jax
import jax.numpy as jnp
from jax import lax
from jax.experimental import pallas as pl
from jax.experimental.pallas import tpu as pltpu
from jax.experimental.pallas import tpu_sc as plsc

N = 10000
E = 160000
D_IN = 256
D_HID = 256
D_OUT2 = 64
N_GRAPHS = 128

NPAD = 10240
K = 128
E_PAD = 16 * 80 * K
NC, NS = 2, 16
ROWS_PER_TILE = NPAD // NS
OUT_ROWS_PER_TILE = N // NS
RB = 400
GRID = N // RB

_mesh = plsc.VectorSubcoreMesh(
    core_axis_name="c", subcore_axis_name="s", num_cores=NC, num_subcores=NS)


def _deg_kernel(col4, ones_hbm, zeros16, out, idx_v, ones_v, deg_sh, sem):
    c = lax.axis_index("c")
    s = lax.axis_index("s")
    r0 = s * ROWS_PER_TILE
    pltpu.sync_copy(zeros16.at[pl.ds(r0, ROWS_PER_TILE)],
                    deg_sh.at[pl.ds(r0, ROWS_PER_TILE)])
    pltpu.sync_copy(ones_hbm, ones_v)
    pltpu.sync_copy(col4.at[c, s], idx_v)
    plsc.subcore_barrier()

    @pl.loop(0, 40)
    def _scatter(j):
        pltpu.sync_copy(ones_v, deg_sh.at[idx_v.at[j]], add=True)

    plsc.subcore_barrier()
    pltpu.sync_copy(deg_sh.at[pl.ds(r0, ROWS_PER_TILE)],
                    out.at[c, pl.ds(r0, ROWS_PER_TILE)])


def _make_deg():
    return pl.kernel(
        _deg_kernel,
        out_type=jax.ShapeDtypeStruct((NC, NPAD, 16), jnp.float32),
        mesh=_mesh,
        scratch_types=[
            pltpu.VMEM((40, K), jnp.int32),
            pltpu.VMEM((K, 16), jnp.float32),
            pltpu.VMEM_SHARED((NPAD, 16), jnp.float32),
            pltpu.SemaphoreType.DMA,
        ],
    )


def _make_agg(W):
    def body(tbl, rows2, col3, zerosw, out, row_v, col_v, buf, agg_sh, sem):
        c = lax.axis_index("c")
        s = lax.axis_index("s")
        r0 = s * ROWS_PER_TILE
        pltpu.sync_copy(zerosw.at[pl.ds(r0, ROWS_PER_TILE)],
                        agg_sh.at[pl.ds(r0, ROWS_PER_TILE)])
        pltpu.sync_copy(rows2.at[c, s], row_v)
        pltpu.sync_copy(col3.at[s], col_v)
        plsc.subcore_barrier()

        @pl.loop(0, 80)
        def _chunk(j):
            pltpu.async_copy(tbl.at[row_v.at[j]], buf, sem).wait()
            pltpu.sync_copy(buf, agg_sh.at[col_v.at[j]], add=True)

        plsc.subcore_barrier()
        o0 = s * OUT_ROWS_PER_TILE
        pltpu.sync_copy(agg_sh.at[pl.ds(o0, OUT_ROWS_PER_TILE)],
                        out.at[c, pl.ds(o0, OUT_ROWS_PER_TILE)])

    return pl.kernel(
        body,
        out_type=jax.ShapeDtypeStruct((NC, N, W), jnp.float32),
        mesh=_mesh,
        scratch_types=[
            pltpu.VMEM((80, K), jnp.int32),
            pltpu.VMEM((80, K), jnp.int32),
            pltpu.VMEM((K, W), jnp.float32),
            pltpu.VMEM_SHARED((NPAD, W), jnp.float32),
            pltpu.SemaphoreType.DMA,
        ],
    )


def _dis_from(deg_ref):
    deg = deg_ref[0, :, 0] + deg_ref[1, :, 0]
    return jnp.where(deg > 0, lax.rsqrt(deg), 0.0)


def _tc1_body(x_ref, w1_ref, v1_ref, b1_ref, deg_ref, hs_ref, xv_ref):
    dis = _dis_from(deg_ref)
    h = jnp.dot(x_ref[...], w1_ref[...], preferred_element_type=jnp.float32)
    hs = h * dis[:, None]
    hs_ref[0] = hs[:, :128]
    hs_ref[1] = hs[:, 128:]
    xv_ref[...] = (jnp.dot(x_ref[...], v1_ref[...],
                           preferred_element_type=jnp.float32)
                   + b1_ref[...])


def _tc2_body(agg_ref, deg_ref, xv_ref, w2_ref, v2_ref, b2_ref,
              hs2_ref, xv2_ref):
    dis = _dis_from(deg_ref)
    agg = jnp.concatenate([agg_ref[0], agg_ref[1]], axis=1)
    out1 = jnp.maximum(agg * dis[:, None] + xv_ref[...], 0.0)
    h2 = jnp.dot(out1, w2_ref[...], preferred_element_type=jnp.float32)
    hs2 = h2 * dis[:, None]
    hs2_ref[0] = hs2[:, :32]
    hs2_ref[1] = hs2[:, 32:]
    xv2_ref[...] = (jnp.dot(out1, v2_ref[...],
                            preferred_element_type=jnp.float32)
                    + b2_ref[...])


def _tc3_body(agg_ref, deg_ref, xv2_ref, batch_ref, fcw_ref, fcb_ref,
              out_ref, acc_sum, acc_cnt):
    i = pl.program_id(0)

    @pl.when(i == 0)
    def _():
        acc_sum[...] = jnp.zeros_like(acc_sum)
        acc_cnt[...] = jnp.zeros_like(acc_cnt)

    dis = _dis_from(deg_ref)
    agg = jnp.concatenate([agg_ref[0], agg_ref[1]], axis=1)
    out2 = jnp.maximum(agg * dis[:, None] + xv2_ref[...], 0.0)
    b = batch_ref[0]
    onehot = (b[:, None] == lax.broadcasted_iota(jnp.int32, (RB, N_GRAPHS), 1)
              ).astype(jnp.float32)
    acc_sum[...] += lax.dot_general(onehot, out2, (((0,), (0,)), ((), ())),
                                    preferred_element_type=jnp.float32)
    acc_cnt[...] += jnp.sum(onehot, axis=0, keepdims=True)

    @pl.when(i == GRID - 1)
    def _():
        pooled = acc_sum[...] / jnp.maximum(acc_cnt[...], 1.0).T
        out_ref[...] = (jnp.dot(pooled, fcw_ref[...],
                                preferred_element_type=jnp.float32)
                        + fcb_ref[...])


def _tc1(x, W1, V1, b1, degS):
    return pl.pallas_call(
        _tc1_body,
        grid=(GRID,),
        in_specs=[
            pl.BlockSpec((RB, D_IN), lambda i: (i, 0)),
            pl.BlockSpec((D_IN, D_HID), lambda i: (0, 0)),
            pl.BlockSpec((D_IN, D_HID), lambda i: (0, 0)),
            pl.BlockSpec((1, D_HID), lambda i: (0, 0)),
            pl.BlockSpec((NC, RB, 16), lambda i: (0, i, 0)),
        ],
        out_specs=[
            pl.BlockSpec((NC, RB, 128), lambda i: (0, i, 0)),
            pl.BlockSpec((RB, D_HID), lambda i: (i, 0)),
        ],
        out_shape=[
            jax.ShapeDtypeStruct((NC, N, 128), jnp.float32),
            jax.ShapeDtypeStruct((N, D_HID), jnp.float32),
        ],
    )(x, W1, V1, b1, degS)


def _tc2(agg1, degS, xv1, W2, V2, b2):
    return pl.pallas_call(
        _tc2_body,
        grid=(GRID,),
        in_specs=[
            pl.BlockSpec((NC, RB, 128), lambda i: (0, i, 0)),
            pl.BlockSpec((NC, RB, 16), lambda i: (0, i, 0)),
            pl.BlockSpec((RB, D_HID), lambda i: (i, 0)),
            pl.BlockSpec((D_HID, D_OUT2), lambda i: (0, 0)),
            pl.BlockSpec((D_HID, D_OUT2), lambda i: (0, 0)),
            pl.BlockSpec((1, D_OUT2), lambda i: (0, 0)),
        ],
        out_specs=[
            pl.BlockSpec((NC, RB, 32), lambda i: (0, i, 0)),
            pl.BlockSpec((RB, D_OUT2), lambda i: (i, 0)),
        ],
        out_shape=[
            jax.ShapeDtypeStruct((NC, N, 32), jnp.float32),
            jax.ShapeDtypeStruct((N, D_OUT2), jnp.float32),
        ],
    )(agg1, degS, xv1, W2, V2, b2)


def _tc3(agg2, degS, xv2, batch2, fc_w, fc_b):
    return pl.pallas_call(
        _tc3_body,
        grid=(GRID,),
        in_specs=[
            pl.BlockSpec((NC, RB, 32), lambda i: (0, i, 0)),
            pl.BlockSpec((NC, RB, 16), lambda i: (0, i, 0)),
            pl.BlockSpec((RB, D_OUT2), lambda i: (i, 0)),
            pl.BlockSpec((1, RB), lambda i: (i, 0)),
            pl.BlockSpec((D_OUT2, 10), lambda i: (0, 0)),
            pl.BlockSpec((1, 10), lambda i: (0, 0)),
        ],
        out_specs=pl.BlockSpec((N_GRAPHS, 10), lambda i: (0, 0)),
        out_shape=jax.ShapeDtypeStruct((N_GRAPHS, 10), jnp.float32),
        scratch_shapes=[
            pltpu.VMEM((N_GRAPHS, D_OUT2), jnp.float32),
            pltpu.VMEM((1, N_GRAPHS), jnp.float32),
        ],
    )(agg2, degS, xv2, batch2, fc_w, fc_b)


def kernel(x, edge_index, batch, W1, V1, b1, W2, V2, b2, fc_w, fc_b):
    ei = edge_index.astype(jnp.int32)
    row, col = ei[0], ei[1]

    pad = E_PAD - E
    spread = jnp.arange(pad, dtype=jnp.int32) % 64
    rowp = jnp.concatenate([row, spread])
    colp = jnp.concatenate([col, 10016 + spread])
    rows2 = jnp.stack([rowp, rowp + N]).reshape(NC, NS, 80, K)
    col3 = colp.reshape(NS, 80, K)
    col4 = colp.reshape(NC, NS, 40, K)
    batch2 = batch.astype(jnp.int32).reshape(GRID, RB)

    ones16 = jnp.ones((K, 16), jnp.float32)
    zeros16 = jnp.zeros((NPAD, 16), jnp.float32)
    zeros128 = jnp.zeros((NPAD, 128), jnp.float32)
    zeros32 = jnp.zeros((NPAD, 32), jnp.float32)

    degS = _make_deg()(col4, ones16, zeros16)

    hs1, xv1 = _tc1(x, W1, V1, b1.reshape(1, D_HID), degS)
    agg1 = _make_agg(128)(hs1.reshape(NC * N, 128), rows2, col3, zeros128)

    hs2, xv2 = _tc2(agg1, degS, xv1, W2, V2, b2.reshape(1, D_OUT2))
    agg2 = _make_agg(32)(hs2.reshape(NC * N, 32), rows2, col3, zeros32)

    return _tc3(agg2, degS, xv2, batch2, fc_w, fc_b)


# speedup vs baseline: 10.1430x; 10.1430x over previous
import functools

import jax
import jax.numpy as jnp
from jax import lax
from jax.experimental import pallas as pl
from jax.experimental.pallas import tpu as pltpu
from jax.experimental.pallas import tpu_sc as plsc

N = 10000
E = 160000
D_IN = 256
D_HID = 256
D_OUT2 = 64
N_GRAPHS = 128

NPAD = 10240
K = 128
E_PAD = 16 * 80 * K
NC, NS = 2, 16
ROWS_PER_TILE = NPAD // NS
RB = 400
GRID = N // RB

@functools.cache
def _mesh():
    return plsc.VectorSubcoreMesh(
        core_axis_name="c", subcore_axis_name="s",
        num_cores=NC, num_subcores=NS)


def _deg_kernel(col4, ones_hbm, zeros128, out, idx_v, ones_v, deg_sh, sem):
    c = lax.axis_index("c")
    s = lax.axis_index("s")
    r0 = s * ROWS_PER_TILE
    pltpu.sync_copy(zeros128.at[pl.ds(r0, ROWS_PER_TILE)],
                    deg_sh.at[pl.ds(r0, ROWS_PER_TILE)])
    pltpu.sync_copy(ones_hbm, ones_v)
    pltpu.sync_copy(col4.at[c, s], idx_v)
    plsc.subcore_barrier()

    @pl.loop(0, 40)
    def _scatter(j):
        pltpu.sync_copy(ones_v, deg_sh.at[idx_v.at[j]], add=True)

    plsc.subcore_barrier()
    pltpu.sync_copy(deg_sh.at[pl.ds(r0, ROWS_PER_TILE)],
                    out.at[c, pl.ds(r0, ROWS_PER_TILE)])


def _make_deg():
    return pl.kernel(
        _deg_kernel,
        out_type=jax.ShapeDtypeStruct((NC, NPAD, 128), jnp.float32),
        mesh=_mesh(),
        scratch_types=[
            pltpu.VMEM((40, K), jnp.int32),
            pltpu.VMEM((K, 128), jnp.float32),
            pltpu.VMEM_SHARED((NPAD, 128), jnp.float32),
            pltpu.SemaphoreType.DMA,
        ],
    )


def _make_agg(W):
    def body(tbl, rows2, col3, zerosw, out, row_v, col_v, buf, agg_sh, sem):
        c = lax.axis_index("c")
        s = lax.axis_index("s")
        r0 = s * ROWS_PER_TILE
        pltpu.sync_copy(zerosw.at[pl.ds(r0, ROWS_PER_TILE)],
                        agg_sh.at[pl.ds(r0, ROWS_PER_TILE)])
        pltpu.sync_copy(rows2.at[c, s], row_v)
        pltpu.sync_copy(col3.at[s], col_v)
        plsc.subcore_barrier()

        @pl.loop(0, 80)
        def _chunk(j):
            pltpu.async_copy(tbl.at[row_v.at[j]], buf, sem).wait()
            pltpu.sync_copy(buf, agg_sh.at[col_v.at[j]], add=True)

        plsc.subcore_barrier()
        pltpu.sync_copy(agg_sh.at[pl.ds(r0, ROWS_PER_TILE)],
                        out.at[c, pl.ds(r0, ROWS_PER_TILE)])

    return pl.kernel(
        body,
        out_type=jax.ShapeDtypeStruct((NC, NPAD, W), jnp.float32),
        mesh=_mesh(),
        scratch_types=[
            pltpu.VMEM((80, K), jnp.int32),
            pltpu.VMEM((80, K), jnp.int32),
            pltpu.VMEM((K, W), jnp.float32),
            pltpu.VMEM_SHARED((NPAD, W), jnp.float32),
            pltpu.SemaphoreType.DMA,
        ],
    )


def _make_agg2():
    def body(tbl, row4, col4, zerosw, out, row_v, col_v, buf, agg_sh, sem):
        c = lax.axis_index("c")
        s = lax.axis_index("s")
        r0 = s * ROWS_PER_TILE
        pltpu.sync_copy(zerosw.at[pl.ds(r0, ROWS_PER_TILE)],
                        agg_sh.at[pl.ds(r0, ROWS_PER_TILE)])
        pltpu.sync_copy(row4.at[c, s], row_v)
        pltpu.sync_copy(col4.at[c, s], col_v)
        plsc.subcore_barrier()

        @pl.loop(0, 40)
        def _chunk(j):
            pltpu.async_copy(tbl.at[row_v.at[j]], buf, sem).wait()
            pltpu.sync_copy(buf, agg_sh.at[col_v.at[j]], add=True)

        plsc.subcore_barrier()
        pltpu.sync_copy(agg_sh.at[pl.ds(r0, ROWS_PER_TILE)],
                        out.at[c, pl.ds(r0, ROWS_PER_TILE)])

    return pl.kernel(
        body,
        out_type=jax.ShapeDtypeStruct((NC, NPAD, 128), jnp.float32),
        mesh=_mesh(),
        scratch_types=[
            pltpu.VMEM((40, K), jnp.int32),
            pltpu.VMEM((40, K), jnp.int32),
            pltpu.VMEM((K, 128), jnp.float32),
            pltpu.VMEM_SHARED((NPAD, 128), jnp.float32),
            pltpu.SemaphoreType.DMA,
        ],
    )


def _dis_from(deg_ref):
    deg = deg_ref[0, :, 0] + deg_ref[1, :, 0]
    return jnp.where(deg > 0, lax.rsqrt(deg), 0.0)


def _tc1_body(x_ref, w1_ref, v1_ref, b1_ref, deg_ref, hs_ref, xv_ref):
    dis = _dis_from(deg_ref)
    h = jnp.dot(x_ref[...], w1_ref[...], preferred_element_type=jnp.float32)
    hs = h * dis[:, None]
    hs_ref[0] = hs[:, :128]
    hs_ref[1] = hs[:, 128:]
    xv_ref[...] = (jnp.dot(x_ref[...], v1_ref[...],
                           preferred_element_type=jnp.float32)
                   + b1_ref[...])


def _tc2_body(agg_ref, deg_ref, xv_ref, w2_ref, v2_ref, b2_ref,
              hs2_ref, xv2_ref):
    dis = _dis_from(deg_ref)
    agg = jnp.concatenate([agg_ref[0], agg_ref[1]], axis=1)
    out1 = jnp.maximum(agg * dis[:, None] + xv_ref[...], 0.0)
    h2 = jnp.dot(out1, w2_ref[...], preferred_element_type=jnp.float32)
    hs2 = h2 * dis[:, None]
    hs2_ref[...] = jnp.concatenate(
        [hs2, jnp.zeros((RB, 128 - D_OUT2), jnp.float32)], axis=1)
    xv2_ref[...] = (jnp.dot(out1, v2_ref[...],
                            preferred_element_type=jnp.float32)
                    + b2_ref[...])


def _tc3_body(agg_ref, deg_ref, xv2_ref, batch_ref, fcw_ref, fcb_ref,
              out_ref, acc_sum, acc_cnt):
    i = pl.program_id(0)

    @pl.when(i == 0)
    def _():
        acc_sum[...] = jnp.zeros_like(acc_sum)
        acc_cnt[...] = jnp.zeros_like(acc_cnt)

    dis = _dis_from(deg_ref)
    agg = agg_ref[0, :, :D_OUT2] + agg_ref[1, :, :D_OUT2]
    out2 = jnp.maximum(agg * dis[:, None] + xv2_ref[...], 0.0)
    b = batch_ref[0, 0]
    onehot = (b[:, None] == lax.broadcasted_iota(jnp.int32, (RB, N_GRAPHS), 1)
              ).astype(jnp.float32)
    acc_sum[...] += lax.dot_general(onehot, out2, (((0,), (0,)), ((), ())),
                                    preferred_element_type=jnp.float32)
    acc_cnt[...] += jnp.sum(onehot, axis=0, keepdims=True)

    @pl.when(i == GRID - 1)
    def _():
        pooled = acc_sum[...] / jnp.maximum(acc_cnt[...], 1.0).T
        out_ref[...] = (jnp.dot(pooled, fcw_ref[...],
                                preferred_element_type=jnp.float32)
                        + fcb_ref[...])


def _tc1(x, W1, V1, b1, degS):
    return pl.pallas_call(
        _tc1_body,
        grid=(GRID,),
        in_specs=[
            pl.BlockSpec((RB, D_IN), lambda i: (i, 0)),
            pl.BlockSpec((D_IN, D_HID), lambda i: (0, 0)),
            pl.BlockSpec((D_IN, D_HID), lambda i: (0, 0)),
            pl.BlockSpec((1, D_HID), lambda i: (0, 0)),
            pl.BlockSpec((NC, RB, 128), lambda i: (0, i, 0)),
        ],
        out_specs=[
            pl.BlockSpec((NC, RB, 128), lambda i: (0, i, 0)),
            pl.BlockSpec((RB, D_HID), lambda i: (i, 0)),
        ],
        out_shape=[
            jax.ShapeDtypeStruct((NC, N, 128), jnp.float32),
            jax.ShapeDtypeStruct((N, D_HID), jnp.float32),
        ],
    )(x, W1, V1, b1, degS)


def _tc2(agg1, degS, xv1, W2, V2, b2):
    return pl.pallas_call(
        _tc2_body,
        grid=(GRID,),
        in_specs=[
            pl.BlockSpec((NC, RB, 128), lambda i: (0, i, 0)),
            pl.BlockSpec((NC, RB, 128), lambda i: (0, i, 0)),
            pl.BlockSpec((RB, D_HID), lambda i: (i, 0)),
            pl.BlockSpec((D_HID, D_OUT2), lambda i: (0, 0)),
            pl.BlockSpec((D_HID, D_OUT2), lambda i: (0, 0)),
            pl.BlockSpec((1, D_OUT2), lambda i: (0, 0)),
        ],
        out_specs=[
            pl.BlockSpec((RB, 128), lambda i: (i, 0)),
            pl.BlockSpec((RB, D_OUT2), lambda i: (i, 0)),
        ],
        out_shape=[
            jax.ShapeDtypeStruct((N, 128), jnp.float32),
            jax.ShapeDtypeStruct((N, D_OUT2), jnp.float32),
        ],
    )(agg1, degS, xv1, W2, V2, b2)


def _tc3(agg2, degS, xv2, batch2, fc_w, fc_b):
    return pl.pallas_call(
        _tc3_body,
        grid=(GRID,),
        in_specs=[
            pl.BlockSpec((NC, RB, 128), lambda i: (0, i, 0)),
            pl.BlockSpec((NC, RB, 128), lambda i: (0, i, 0)),
            pl.BlockSpec((RB, D_OUT2), lambda i: (i, 0)),
            pl.BlockSpec((1, 1, RB), lambda i: (i, 0, 0)),
            pl.BlockSpec((D_OUT2, 10), lambda i: (0, 0)),
            pl.BlockSpec((1, 10), lambda i: (0, 0)),
        ],
        out_specs=pl.BlockSpec((N_GRAPHS, 10), lambda i: (0, 0)),
        out_shape=jax.ShapeDtypeStruct((N_GRAPHS, 10), jnp.float32),
        scratch_shapes=[
            pltpu.VMEM((N_GRAPHS, D_OUT2), jnp.float32),
            pltpu.VMEM((1, N_GRAPHS), jnp.float32),
        ],
    )(agg2, degS, xv2, batch2, fc_w, fc_b)


def kernel(x, edge_index, batch, W1, V1, b1, W2, V2, b2, fc_w, fc_b):
    ei = edge_index.astype(jnp.int32)
    row, col = ei[0], ei[1]

    pad = E_PAD - E
    spread = jnp.arange(pad, dtype=jnp.int32) % 64
    rowp = jnp.concatenate([row, spread])
    colp = jnp.concatenate([col, 10016 + spread])
    rows2 = jnp.stack([rowp, rowp + N]).reshape(NC, NS, 80, K)
    col3 = colp.reshape(NS, 80, K)
    col4 = colp.reshape(NC, NS, 40, K)
    row4 = rowp.reshape(NC, NS, 40, K)
    batch2 = batch.astype(jnp.int32).reshape(GRID, 1, RB)

    ones128 = jnp.ones((K, 128), jnp.float32)
    zeros128 = jnp.zeros((NPAD, 128), jnp.float32)

    degS = _make_deg()(col4, ones128, zeros128)

    hs1, xv1 = _tc1(x, W1, V1, b1.reshape(1, D_HID), degS)
    agg1 = _make_agg(128)(hs1.reshape(NC * N, 128), rows2, col3, zeros128)

    hs2, xv2 = _tc2(agg1, degS, xv1, W2, V2, b2.reshape(1, D_OUT2))
    agg2 = _make_agg2()(hs2, row4, col4, zeros128)

    return _tc3(agg2, degS, xv2, batch2, fc_w, fc_b.reshape(1, 10))


# speedup vs baseline: 13.4191x; 1.3230x over previous
import functools

import jax
import jax.numpy as jnp
from jax import lax
from jax.experimental import pallas as pl
from jax.experimental.pallas import tpu as pltpu
from jax.experimental.pallas import tpu_sc as plsc

N = 10000
E = 160000
D_IN = 256
D_HID = 256
D_OUT2 = 64
N_GRAPHS = 128

NPAD = 10240
K = 128
E_PAD = 163840
NC, NS = 2, 16
ROWS_PER_TILE = NPAD // NS
C1 = E_PAD // (NS * K)
C2 = E_PAD // (NC * NS * K)
RB = 400
GRID = N // RB


@functools.cache
def _mesh():
    return plsc.VectorSubcoreMesh(
        core_axis_name="c", subcore_axis_name="s",
        num_cores=NC, num_subcores=NS)


def _deg_kernel(col4, ones_hbm, zeros128, out, idx_v, ones_v, deg_sh, sem):
    c = lax.axis_index("c")
    s = lax.axis_index("s")
    r0 = s * ROWS_PER_TILE
    pltpu.sync_copy(zeros128.at[pl.ds(r0, ROWS_PER_TILE)],
                    deg_sh.at[pl.ds(r0, ROWS_PER_TILE)])
    pltpu.sync_copy(ones_hbm, ones_v)
    pltpu.sync_copy(col4.at[c, s], idx_v)
    plsc.subcore_barrier()

    @pl.loop(0, C2)
    def _scatter(j):
        pltpu.sync_copy(ones_v, deg_sh.at[idx_v.at[j]], add=True)

    plsc.subcore_barrier()
    pltpu.sync_copy(deg_sh.at[pl.ds(r0, ROWS_PER_TILE)],
                    out.at[c, pl.ds(r0, ROWS_PER_TILE)])


def _make_deg():
    return pl.kernel(
        _deg_kernel,
        out_type=jax.ShapeDtypeStruct((NC, NPAD, 128), jnp.float32),
        mesh=_mesh(),
        scratch_types=[
            pltpu.VMEM((C2, K), jnp.int32),
            pltpu.VMEM((K, 128), jnp.float32),
            pltpu.VMEM_SHARED((NPAD, 128), jnp.float32),
            pltpu.SemaphoreType.DMA,
        ],
    )


def _pipelined_edges(tbl, rows_hbm, ring, col_v, bufs, agg_sh,
                     gsem, ssem, isem, chunks):
    assert chunks % 8 == 0

    for q in range(4):
        pltpu.async_copy(rows_hbm.at[q], ring.at[q], isem[q])

    @pl.loop(0, chunks // 8)
    def _outer(g):
        for jj in range(8):
            j = g * 8 + jj
            b, q, bp = jj % 2, jj % 4, (jj - 1) % 2

            def _free_and_refill(j=j, b=b, q=q):
                pltpu.make_async_copy(
                    bufs[b], agg_sh.at[col_v.at[j]], ssem[b]).wait()

                @pl.when(j < chunks - 2)
                def _():
                    pltpu.async_copy(rows_hbm.at[j + 2],
                                     ring.at[(q + 2) % 4], isem[(q + 2) % 4])

            if jj < 2:
                @pl.when(g > 0)
                def _(f=_free_and_refill):
                    f()
            else:
                _free_and_refill()

            pltpu.make_async_copy(rows_hbm.at[j], ring.at[q], isem[q]).wait()
            pltpu.async_copy(tbl.at[ring.at[q]], bufs[b], gsem[b])

            def _scatter_prev(j=j, bp=bp):
                pltpu.make_async_copy(
                    tbl.at[ring.at[0]], bufs[bp], gsem[bp]).wait()
                pltpu.async_copy(bufs[bp], agg_sh.at[col_v.at[j - 1]],
                                 ssem[bp], add=True)

            if jj == 0:
                @pl.when(g > 0)
                def _(f=_scatter_prev):
                    f()
            else:
                _scatter_prev()

    last = chunks - 1
    pltpu.make_async_copy(tbl.at[ring.at[0]], bufs[last % 2],
                          gsem[last % 2]).wait()
    pltpu.async_copy(bufs[last % 2], agg_sh.at[col_v.at[last]],
                     ssem[last % 2], add=True)
    for b in range(2):
        pltpu.make_async_copy(bufs[b], agg_sh.at[col_v.at[0]],
                              ssem[b]).wait()


def _make_agg(W):
    def body(tbl, rows2, col3, zerosw, out, col_v, ring, b0, b1,
             agg_sh, g0, g1, s0, s1, i0, i1, i2, i3):
        c = lax.axis_index("c")
        s = lax.axis_index("s")
        r0 = s * ROWS_PER_TILE
        pltpu.sync_copy(zerosw.at[pl.ds(r0, ROWS_PER_TILE)],
                        agg_sh.at[pl.ds(r0, ROWS_PER_TILE)])
        pltpu.sync_copy(col3.at[s], col_v)
        plsc.subcore_barrier()
        _pipelined_edges(tbl, rows2.at[c, s], ring, col_v, [b0, b1], agg_sh,
                         [g0, g1], [s0, s1], [i0, i1, i2, i3], C1)
        plsc.subcore_barrier()
        pltpu.sync_copy(agg_sh.at[pl.ds(r0, ROWS_PER_TILE)],
                        out.at[c, pl.ds(r0, ROWS_PER_TILE)])

    return pl.kernel(
        body,
        out_type=jax.ShapeDtypeStruct((NC, NPAD, W), jnp.float32),
        mesh=_mesh(),
        scratch_types=[
            pltpu.VMEM((C1, K), jnp.int32),
            pltpu.VMEM((4, K), jnp.int32),
            pltpu.VMEM((K, W), jnp.float32),
            pltpu.VMEM((K, W), jnp.float32),
            pltpu.VMEM_SHARED((NPAD, W), jnp.float32),
            pltpu.SemaphoreType.DMA,
            pltpu.SemaphoreType.DMA,
            pltpu.SemaphoreType.DMA,
            pltpu.SemaphoreType.DMA,
            pltpu.SemaphoreType.DMA,
            pltpu.SemaphoreType.DMA,
            pltpu.SemaphoreType.DMA,
            pltpu.SemaphoreType.DMA,
        ],
    )


def _make_agg2():
    def body(tbl, row4, col4, zerosw, out, col_v, ring, b0, b1,
             agg_sh, g0, g1, s0, s1, i0, i1, i2, i3):
        c = lax.axis_index("c")
        s = lax.axis_index("s")
        r0 = s * ROWS_PER_TILE
        pltpu.sync_copy(zerosw.at[pl.ds(r0, ROWS_PER_TILE)],
                        agg_sh.at[pl.ds(r0, ROWS_PER_TILE)])
        pltpu.sync_copy(col4.at[c, s], col_v)
        plsc.subcore_barrier()
        _pipelined_edges(tbl, row4.at[c, s], ring, col_v, [b0, b1], agg_sh,
                         [g0, g1], [s0, s1], [i0, i1, i2, i3], C2)
        plsc.subcore_barrier()
        pltpu.sync_copy(agg_sh.at[pl.ds(r0, ROWS_PER_TILE)],
                        out.at[c, pl.ds(r0, ROWS_PER_TILE)])

    return pl.kernel(
        body,
        out_type=jax.ShapeDtypeStruct((NC, NPAD, 128), jnp.float32),
        mesh=_mesh(),
        scratch_types=[
            pltpu.VMEM((C2, K), jnp.int32),
            pltpu.VMEM((4, K), jnp.int32),
            pltpu.VMEM((K, 128), jnp.float32),
            pltpu.VMEM((K, 128), jnp.float32),
            pltpu.VMEM_SHARED((NPAD, 128), jnp.float32),
            pltpu.SemaphoreType.DMA,
            pltpu.SemaphoreType.DMA,
            pltpu.SemaphoreType.DMA,
            pltpu.SemaphoreType.DMA,
            pltpu.SemaphoreType.DMA,
            pltpu.SemaphoreType.DMA,
            pltpu.SemaphoreType.DMA,
            pltpu.SemaphoreType.DMA,
        ],
    )


def _dis_from(deg_ref):
    deg = deg_ref[0, 0, 0] + deg_ref[1, 0, 0]
    return jnp.where(deg > 0, lax.rsqrt(deg), 0.0)


def _tc1_body(x_ref, w1_ref, v1_ref, b1_ref, deg_ref, hs_ref, xv_ref):
    dis = _dis_from(deg_ref)
    h = jnp.dot(x_ref[...], w1_ref[...], preferred_element_type=jnp.float32)
    hs = h * dis[:, None]
    hs_ref[0] = hs[:, :128]
    hs_ref[1] = hs[:, 128:]
    xv_ref[...] = (jnp.dot(x_ref[...], v1_ref[...],
                           preferred_element_type=jnp.float32)
                   + b1_ref[...])


def _tc2_body(agg_ref, deg_ref, xv_ref, w2_ref, v2_ref, b2_ref,
              hs2_ref, xv2_ref):
    dis = _dis_from(deg_ref)
    agg = jnp.concatenate([agg_ref[0], agg_ref[1]], axis=1)
    out1 = jnp.maximum(agg * dis[:, None] + xv_ref[...], 0.0)
    h2 = jnp.dot(out1, w2_ref[...], preferred_element_type=jnp.float32)
    hs2 = h2 * dis[:, None]
    hs2_ref[...] = jnp.concatenate(
        [hs2, jnp.zeros((RB, 128 - D_OUT2), jnp.float32)], axis=1)
    xv2_ref[...] = (jnp.dot(out1, v2_ref[...],
                            preferred_element_type=jnp.float32)
                    + b2_ref[...])


def _tc3_body(agg_ref, deg_ref, xv2_ref, batch_ref, fcw_ref, fcb_ref,
              out_ref, acc_sum, acc_cnt):
    i = pl.program_id(0)

    @pl.when(i == 0)
    def _():
        acc_sum[...] = jnp.zeros_like(acc_sum)
        acc_cnt[...] = jnp.zeros_like(acc_cnt)

    dis = _dis_from(deg_ref)
    agg = agg_ref[0, :, :D_OUT2] + agg_ref[1, :, :D_OUT2]
    out2 = jnp.maximum(agg * dis[:, None] + xv2_ref[...], 0.0)
    b = batch_ref[0, 0]
    onehot = (b[:, None] == lax.broadcasted_iota(jnp.int32, (RB, N_GRAPHS), 1)
              ).astype(jnp.float32)
    acc_sum[...] += lax.dot_general(onehot, out2, (((0,), (0,)), ((), ())),
                                    preferred_element_type=jnp.float32)
    acc_cnt[...] += jnp.sum(onehot, axis=0, keepdims=True)

    @pl.when(i == GRID - 1)
    def _():
        pooled = acc_sum[...] / jnp.maximum(acc_cnt[...], 1.0).T
        out_ref[...] = (jnp.dot(pooled, fcw_ref[...],
                                preferred_element_type=jnp.float32)
                        + fcb_ref[...])


def _tc1(x, W1, V1, b1, degS):
    return pl.pallas_call(
        _tc1_body,
        grid=(GRID,),
        in_specs=[
            pl.BlockSpec((RB, D_IN), lambda i: (i, 0)),
            pl.BlockSpec((D_IN, D_HID), lambda i: (0, 0)),
            pl.BlockSpec((D_IN, D_HID), lambda i: (0, 0)),
            pl.BlockSpec((1, D_HID), lambda i: (0, 0)),
            pl.BlockSpec((NC, 1, 1, RB), lambda i: (0, i, 0, 0)),
        ],
        out_specs=[
            pl.BlockSpec((NC, RB, 128), lambda i: (0, i, 0)),
            pl.BlockSpec((RB, D_HID), lambda i: (i, 0)),
        ],
        out_shape=[
            jax.ShapeDtypeStruct((NC, N, 128), jnp.float32),
            jax.ShapeDtypeStruct((N, D_HID), jnp.float32),
        ],
    )(x, W1, V1, b1, degS)


def _tc2(agg1, degS, xv1, W2, V2, b2):
    return pl.pallas_call(
        _tc2_body,
        grid=(GRID,),
        in_specs=[
            pl.BlockSpec((NC, RB, 128), lambda i: (0, i, 0)),
            pl.BlockSpec((NC, 1, 1, RB), lambda i: (0, i, 0, 0)),
            pl.BlockSpec((RB, D_HID), lambda i: (i, 0)),
            pl.BlockSpec((D_HID, D_OUT2), lambda i: (0, 0)),
            pl.BlockSpec((D_HID, D_OUT2), lambda i: (0, 0)),
            pl.BlockSpec((1, D_OUT2), lambda i: (0, 0)),
        ],
        out_specs=[
            pl.BlockSpec((RB, 128), lambda i: (i, 0)),
            pl.BlockSpec((RB, D_OUT2), lambda i: (i, 0)),
        ],
        out_shape=[
            jax.ShapeDtypeStruct((N, 128), jnp.float32),
            jax.ShapeDtypeStruct((N, D_OUT2), jnp.float32),
        ],
    )(agg1, degS, xv1, W2, V2, b2)


def _tc3(agg2, degS, xv2, batch2, fc_w, fc_b):
    return pl.pallas_call(
        _tc3_body,
        grid=(GRID,),
        in_specs=[
            pl.BlockSpec((NC, RB, 128), lambda i: (0, i, 0)),
            pl.BlockSpec((NC, 1, 1, RB), lambda i: (0, i, 0, 0)),
            pl.BlockSpec((RB, D_OUT2), lambda i: (i, 0)),
            pl.BlockSpec((1, 1, RB), lambda i: (i, 0, 0)),
            pl.BlockSpec((D_OUT2, 10), lambda i: (0, 0)),
            pl.BlockSpec((1, 10), lambda i: (0, 0)),
        ],
        out_specs=pl.BlockSpec((N_GRAPHS, 10), lambda i: (0, 0)),
        out_shape=jax.ShapeDtypeStruct((N_GRAPHS, 10), jnp.float32),
        scratch_shapes=[
            pltpu.VMEM((N_GRAPHS, D_OUT2), jnp.float32),
            pltpu.VMEM((1, N_GRAPHS), jnp.float32),
        ],
    )(agg2, degS, xv2, batch2, fc_w, fc_b)


def kernel(x, edge_index, batch, W1, V1, b1, W2, V2, b2, fc_w, fc_b):
    ei = edge_index.astype(jnp.int32)
    row, col = ei[0], ei[1]

    pad = E_PAD - E
    spread = jnp.arange(pad, dtype=jnp.int32) % 64
    rowp = jnp.concatenate([row, spread])
    colp = jnp.concatenate([col, 10016 + spread])
    rows2 = jnp.stack([rowp, rowp + N]).reshape(NC, NS, C1, K)
    col3 = colp.reshape(NS, C1, K)
    col4 = colp.reshape(NC, NS, C2, K)
    row4 = rowp.reshape(NC, NS, C2, K)
    batch2 = batch.astype(jnp.int32).reshape(GRID, 1, RB)

    ones128 = jnp.ones((K, 128), jnp.float32)
    zeros128 = jnp.zeros((NPAD, 128), jnp.float32)

    degS = _make_deg()(col4, ones128, zeros128)[:, :N, 0].reshape(
        NC, GRID, 1, RB)

    hs1, xv1 = _tc1(x, W1, V1, b1.reshape(1, D_HID), degS)
    agg1 = _make_agg(128)(hs1.reshape(NC * N, 128), rows2, col3, zeros128)

    hs2, xv2 = _tc2(agg1, degS, xv1, W2, V2, b2.reshape(1, D_OUT2))
    agg2 = _make_agg2()(hs2, row4, col4, zeros128)

    return _tc3(agg2, degS, xv2, batch2, fc_w, fc_b.reshape(1, 10))


# speedup vs baseline: 13.9251x; 1.0377x over previous
import functools

import jax
import jax.numpy as jnp
from jax import lax
from jax.experimental import pallas as pl
from jax.experimental.pallas import tpu as pltpu
from jax.experimental.pallas import tpu_sc as plsc

N = 10000
E = 160000
D_IN = 256
D_HID = 256
D_OUT2 = 64
N_GRAPHS = 128

NPAD = 10240
K = 128
E_PAD = 163840
NC, NS = 2, 16
ROWS_PER_TILE = NPAD // NS
C1 = E_PAD // (NS * K)
C2 = E_PAD // (NC * NS * K)
RB = 400
GRID = N // RB


@functools.cache
def _mesh():
    return plsc.VectorSubcoreMesh(
        core_axis_name="c", subcore_axis_name="s",
        num_cores=NC, num_subcores=NS)


ZROWS = 32


def _zero_spmem_slice(zbuf, agg_sh, r0):
    zeros16 = jnp.zeros((16,), jnp.float32)

    @pl.loop(0, ZROWS)
    def _zrow(i):
        for kk in range(8):
            zbuf[i, pl.ds(kk * 16, 16)] = zeros16

    @pl.loop(0, ROWS_PER_TILE // ZROWS)
    def _zcopy(t):
        pltpu.sync_copy(zbuf, agg_sh.at[pl.ds(r0 + t * ZROWS, ZROWS)])


def _deg_kernel(col4, ones_hbm, out, idx_v, ones_v, zbuf, deg_sh, sem):
    c = lax.axis_index("c")
    s = lax.axis_index("s")
    r0 = s * ROWS_PER_TILE
    _zero_spmem_slice(zbuf, deg_sh, r0)
    pltpu.sync_copy(ones_hbm, ones_v)
    pltpu.sync_copy(col4.at[c, s], idx_v)
    plsc.subcore_barrier()

    @pl.loop(0, C2)
    def _scatter(j):
        pltpu.sync_copy(ones_v, deg_sh.at[idx_v.at[j]], add=True)

    plsc.subcore_barrier()
    pltpu.sync_copy(deg_sh.at[pl.ds(r0, ROWS_PER_TILE)],
                    out.at[c, pl.ds(r0, ROWS_PER_TILE)])


def _make_deg():
    return pl.kernel(
        _deg_kernel,
        out_type=jax.ShapeDtypeStruct((NC, NPAD, 128), jnp.float32),
        mesh=_mesh(),
        scratch_types=[
            pltpu.VMEM((C2, K), jnp.int32),
            pltpu.VMEM((K, 128), jnp.float32),
            pltpu.VMEM((ZROWS, 128), jnp.float32),
            pltpu.VMEM_SHARED((NPAD, 128), jnp.float32),
            pltpu.SemaphoreType.DMA,
        ],
    )


def _pipelined_edges(tbl, rows_hbm, ring, col_v, bufs, agg_sh,
                     gsem, ssem, isem, chunks):
    assert chunks % 8 == 0

    for q in range(4):
        pltpu.async_copy(rows_hbm.at[q], ring.at[q], isem[q])

    @pl.loop(0, chunks // 8)
    def _outer(g):
        for jj in range(8):
            j = g * 8 + jj
            b, q, bp = jj % 2, jj % 4, (jj - 1) % 2

            def _free_and_refill(j=j, b=b, q=q):
                pltpu.make_async_copy(
                    bufs[b], agg_sh.at[col_v.at[j]], ssem[b]).wait()

                @pl.when(j < chunks - 2)
                def _():
                    pltpu.async_copy(rows_hbm.at[j + 2],
                                     ring.at[(q + 2) % 4], isem[(q + 2) % 4])

            if jj < 2:
                @pl.when(g > 0)
                def _(f=_free_and_refill):
                    f()
            else:
                _free_and_refill()

            pltpu.make_async_copy(rows_hbm.at[j], ring.at[q], isem[q]).wait()
            pltpu.async_copy(tbl.at[ring.at[q]], bufs[b], gsem[b])

            def _scatter_prev(j=j, bp=bp):
                pltpu.make_async_copy(
                    tbl.at[ring.at[0]], bufs[bp], gsem[bp]).wait()
                pltpu.async_copy(bufs[bp], agg_sh.at[col_v.at[j - 1]],
                                 ssem[bp], add=True)

            if jj == 0:
                @pl.when(g > 0)
                def _(f=_scatter_prev):
                    f()
            else:
                _scatter_prev()

    last = chunks - 1
    pltpu.make_async_copy(tbl.at[ring.at[0]], bufs[last % 2],
                          gsem[last % 2]).wait()
    pltpu.async_copy(bufs[last % 2], agg_sh.at[col_v.at[last]],
                     ssem[last % 2], add=True)
    for b in range(2):
        pltpu.make_async_copy(bufs[b], agg_sh.at[col_v.at[0]],
                              ssem[b]).wait()


def _make_agg(W):
    def body(tbl, rows2, col3, out, col_v, ring, b0, b1, zbuf,
             agg_sh, g0, g1, s0, s1, i0, i1, i2, i3):
        c = lax.axis_index("c")
        s = lax.axis_index("s")
        r0 = s * ROWS_PER_TILE
        _zero_spmem_slice(zbuf, agg_sh, r0)
        pltpu.sync_copy(col3.at[s], col_v)
        plsc.subcore_barrier()
        _pipelined_edges(tbl, rows2.at[c, s], ring, col_v, [b0, b1], agg_sh,
                         [g0, g1], [s0, s1], [i0, i1, i2, i3], C1)
        plsc.subcore_barrier()
        pltpu.sync_copy(agg_sh.at[pl.ds(r0, ROWS_PER_TILE)],
                        out.at[c, pl.ds(r0, ROWS_PER_TILE)])

    return pl.kernel(
        body,
        out_type=jax.ShapeDtypeStruct((NC, NPAD, W), jnp.float32),
        mesh=_mesh(),
        scratch_types=[
            pltpu.VMEM((C1, K), jnp.int32),
            pltpu.VMEM((4, K), jnp.int32),
            pltpu.VMEM((K, W), jnp.float32),
            pltpu.VMEM((K, W), jnp.float32),
            pltpu.VMEM((ZROWS, 128), jnp.float32),
            pltpu.VMEM_SHARED((NPAD, W), jnp.float32),
            pltpu.SemaphoreType.DMA,
            pltpu.SemaphoreType.DMA,
            pltpu.SemaphoreType.DMA,
            pltpu.SemaphoreType.DMA,
            pltpu.SemaphoreType.DMA,
            pltpu.SemaphoreType.DMA,
            pltpu.SemaphoreType.DMA,
            pltpu.SemaphoreType.DMA,
        ],
    )


def _make_agg2():
    def body(tbl, row4, col4, out, col_v, ring, b0, b1, zbuf,
             agg_sh, g0, g1, s0, s1, i0, i1, i2, i3):
        c = lax.axis_index("c")
        s = lax.axis_index("s")
        r0 = s * ROWS_PER_TILE
        _zero_spmem_slice(zbuf, agg_sh, r0)
        pltpu.sync_copy(col4.at[c, s], col_v)
        plsc.subcore_barrier()
        _pipelined_edges(tbl, row4.at[c, s], ring, col_v, [b0, b1], agg_sh,
                         [g0, g1], [s0, s1], [i0, i1, i2, i3], C2)
        plsc.subcore_barrier()
        pltpu.sync_copy(agg_sh.at[pl.ds(r0, ROWS_PER_TILE)],
                        out.at[c, pl.ds(r0, ROWS_PER_TILE)])

    return pl.kernel(
        body,
        out_type=jax.ShapeDtypeStruct((NC, NPAD, 128), jnp.float32),
        mesh=_mesh(),
        scratch_types=[
            pltpu.VMEM((C2, K), jnp.int32),
            pltpu.VMEM((4, K), jnp.int32),
            pltpu.VMEM((K, 128), jnp.float32),
            pltpu.VMEM((K, 128), jnp.float32),
            pltpu.VMEM((ZROWS, 128), jnp.float32),
            pltpu.VMEM_SHARED((NPAD, 128), jnp.float32),
            pltpu.SemaphoreType.DMA,
            pltpu.SemaphoreType.DMA,
            pltpu.SemaphoreType.DMA,
            pltpu.SemaphoreType.DMA,
            pltpu.SemaphoreType.DMA,
            pltpu.SemaphoreType.DMA,
            pltpu.SemaphoreType.DMA,
            pltpu.SemaphoreType.DMA,
        ],
    )


def _tc1_body(x_ref, w1_ref, v1_ref, b1_ref, deg_ref, hs_ref, xv_ref,
              dis_ref):
    deg = deg_ref[0, :, 0] + deg_ref[1, :, 0]
    dis = jnp.where(deg > 0, lax.rsqrt(deg), 0.0)
    dis_ref[0, 0] = dis
    h = jnp.dot(x_ref[...], w1_ref[...], preferred_element_type=jnp.float32)
    hs = h * dis[:, None]
    hs_ref[0] = hs[:, :128]
    hs_ref[1] = hs[:, 128:]
    xv_ref[...] = (jnp.dot(x_ref[...], v1_ref[...],
                           preferred_element_type=jnp.float32)
                   + b1_ref[...])


def _tc2_body(agg_ref, dis_ref, xv_ref, w2_ref, v2_ref, b2_ref,
              hs2_ref, xv2_ref):
    dis = dis_ref[0, 0]
    agg = jnp.concatenate([agg_ref[0], agg_ref[1]], axis=1)
    out1 = jnp.maximum(agg * dis[:, None] + xv_ref[...], 0.0)
    h2 = jnp.dot(out1, w2_ref[...], preferred_element_type=jnp.float32)
    hs2 = h2 * dis[:, None]
    hs2_ref[...] = jnp.concatenate(
        [hs2, jnp.zeros((RB, 128 - D_OUT2), jnp.float32)], axis=1)
    xv2_ref[...] = (jnp.dot(out1, v2_ref[...],
                            preferred_element_type=jnp.float32)
                    + b2_ref[...])


def _tc3_body(agg_ref, dis_ref, xv2_ref, batch_ref, fcw_ref, fcb_ref,
              out_ref, acc_sum, acc_cnt):
    i = pl.program_id(0)

    @pl.when(i == 0)
    def _():
        acc_sum[...] = jnp.zeros_like(acc_sum)
        acc_cnt[...] = jnp.zeros_like(acc_cnt)

    dis = dis_ref[0, 0]
    agg = agg_ref[0, :, :D_OUT2] + agg_ref[1, :, :D_OUT2]
    out2 = jnp.maximum(agg * dis[:, None] + xv2_ref[...], 0.0)
    b = batch_ref[0, 0]
    onehot = (b[:, None] == lax.broadcasted_iota(jnp.int32, (RB, N_GRAPHS), 1)
              ).astype(jnp.float32)
    acc_sum[...] += lax.dot_general(onehot, out2, (((0,), (0,)), ((), ())),
                                    preferred_element_type=jnp.float32)
    acc_cnt[...] += jnp.sum(onehot, axis=0, keepdims=True)

    @pl.when(i == GRID - 1)
    def _():
        pooled = acc_sum[...] / jnp.maximum(acc_cnt[...], 1.0).T
        out_ref[...] = (jnp.dot(pooled, fcw_ref[...],
                                preferred_element_type=jnp.float32)
                        + fcb_ref[...])


def _tc1(x, W1, V1, b1, degS):
    return pl.pallas_call(
        _tc1_body,
        grid=(GRID,),
        in_specs=[
            pl.BlockSpec((RB, D_IN), lambda i: (i, 0)),
            pl.BlockSpec((D_IN, D_HID), lambda i: (0, 0)),
            pl.BlockSpec((D_IN, D_HID), lambda i: (0, 0)),
            pl.BlockSpec((1, D_HID), lambda i: (0, 0)),
            pl.BlockSpec((NC, RB, 128), lambda i: (0, i, 0)),
        ],
        out_specs=[
            pl.BlockSpec((NC, RB, 128), lambda i: (0, i, 0)),
            pl.BlockSpec((RB, D_HID), lambda i: (i, 0)),
            pl.BlockSpec((1, 1, RB), lambda i: (i, 0, 0)),
        ],
        out_shape=[
            jax.ShapeDtypeStruct((NC, N, 128), jnp.float32),
            jax.ShapeDtypeStruct((N, D_HID), jnp.float32),
            jax.ShapeDtypeStruct((GRID, 1, RB), jnp.float32),
        ],
    )(x, W1, V1, b1, degS)


def _tc2(agg1, dis3, xv1, W2, V2, b2):
    return pl.pallas_call(
        _tc2_body,
        grid=(GRID,),
        in_specs=[
            pl.BlockSpec((NC, RB, 128), lambda i: (0, i, 0)),
            pl.BlockSpec((1, 1, RB), lambda i: (i, 0, 0)),
            pl.BlockSpec((RB, D_HID), lambda i: (i, 0)),
            pl.BlockSpec((D_HID, D_OUT2), lambda i: (0, 0)),
            pl.BlockSpec((D_HID, D_OUT2), lambda i: (0, 0)),
            pl.BlockSpec((1, D_OUT2), lambda i: (0, 0)),
        ],
        out_specs=[
            pl.BlockSpec((RB, 128), lambda i: (i, 0)),
            pl.BlockSpec((RB, D_OUT2), lambda i: (i, 0)),
        ],
        out_shape=[
            jax.ShapeDtypeStruct((N, 128), jnp.float32),
            jax.ShapeDtypeStruct((N, D_OUT2), jnp.float32),
        ],
    )(agg1, dis3, xv1, W2, V2, b2)


def _tc3(agg2, dis3, xv2, batch2, fc_w, fc_b):
    return pl.pallas_call(
        _tc3_body,
        grid=(GRID,),
        in_specs=[
            pl.BlockSpec((NC, RB, 128), lambda i: (0, i, 0)),
            pl.BlockSpec((1, 1, RB), lambda i: (i, 0, 0)),
            pl.BlockSpec((RB, D_OUT2), lambda i: (i, 0)),
            pl.BlockSpec((1, 1, RB), lambda i: (i, 0, 0)),
            pl.BlockSpec((D_OUT2, 10), lambda i: (0, 0)),
            pl.BlockSpec((1, 10), lambda i: (0, 0)),
        ],
        out_specs=pl.BlockSpec((N_GRAPHS, 10), lambda i: (0, 0)),
        out_shape=jax.ShapeDtypeStruct((N_GRAPHS, 10), jnp.float32),
        scratch_shapes=[
            pltpu.VMEM((N_GRAPHS, D_OUT2), jnp.float32),
            pltpu.VMEM((1, N_GRAPHS), jnp.float32),
        ],
    )(agg2, dis3, xv2, batch2, fc_w, fc_b)


def kernel(x, edge_index, batch, W1, V1, b1, W2, V2, b2, fc_w, fc_b):
    ei = edge_index.astype(jnp.int32)
    row, col = ei[0], ei[1]

    pad = E_PAD - E
    spread = jnp.arange(pad, dtype=jnp.int32) % 64
    rowp = jnp.concatenate([row, spread])
    colp = jnp.concatenate([col, 10016 + spread])
    rows2 = jnp.stack([rowp, rowp + N]).reshape(NC, NS, C1, K)
    col3 = colp.reshape(NS, C1, K)
    col4 = colp.reshape(NC, NS, C2, K)
    row4 = rowp.reshape(NC, NS, C2, K)
    batch2 = batch.astype(jnp.int32).reshape(GRID, 1, RB)

    ones128 = jnp.ones((K, 128), jnp.float32)

    degS = _make_deg()(col4, ones128)

    hs1, xv1, dis3 = _tc1(x, W1, V1, b1.reshape(1, D_HID), degS)
    agg1 = _make_agg(128)(hs1.reshape(NC * N, 128), rows2, col3)

    hs2, xv2 = _tc2(agg1, dis3, xv1, W2, V2, b2.reshape(1, D_OUT2))
    agg2 = _make_agg2()(hs2, row4, col4)

    return _tc3(agg2, dis3, xv2, batch2, fc_w, fc_b.reshape(1, 10))


# speedup vs baseline: 13.9319x; 1.0005x over previous
import functools

import jax
import jax.numpy as jnp
from jax import lax
from jax.experimental import pallas as pl
from jax.experimental.pallas import tpu as pltpu
from jax.experimental.pallas import tpu_sc as plsc

N = 10000
E = 160000
D_IN = 256
D_HID = 256
D_OUT2 = 64
N_GRAPHS = 128

NPAD = 10240
K = 128
E_PAD = 163840
NC, NS = 2, 16
ROWS_PER_TILE = NPAD // NS
C1 = E_PAD // (NS * K)
C2 = E_PAD // (NC * NS * K)
RB = 400
GRID = N // RB


@functools.cache
def _mesh():
    return plsc.VectorSubcoreMesh(
        core_axis_name="c", subcore_axis_name="s",
        num_cores=NC, num_subcores=NS)


ZROWS = 32


def _zero_spmem_slice(zbuf, agg_sh, r0):
    zeros16 = jnp.zeros((16,), jnp.float32)

    @pl.loop(0, ZROWS)
    def _zrow(i):
        for kk in range(8):
            zbuf[i, pl.ds(kk * 16, 16)] = zeros16

    @pl.loop(0, ROWS_PER_TILE // ZROWS)
    def _zcopy(t):
        pltpu.sync_copy(zbuf, agg_sh.at[pl.ds(r0 + t * ZROWS, ZROWS)])


def _deg_kernel(col4, ones_hbm, out, idx_v, ones_v, zbuf, deg_sh, sem):
    c = lax.axis_index("c")
    s = lax.axis_index("s")
    r0 = s * ROWS_PER_TILE
    _zero_spmem_slice(zbuf, deg_sh, r0)
    pltpu.sync_copy(ones_hbm, ones_v)
    pltpu.sync_copy(col4.at[c, s], idx_v)
    plsc.subcore_barrier()

    @pl.loop(0, C2)
    def _scatter(j):
        pltpu.sync_copy(ones_v, deg_sh.at[idx_v.at[j]], add=True)

    plsc.subcore_barrier()
    pltpu.sync_copy(deg_sh.at[pl.ds(r0, ROWS_PER_TILE)],
                    out.at[c, pl.ds(r0, ROWS_PER_TILE)])


def _make_deg():
    return pl.kernel(
        _deg_kernel,
        out_type=jax.ShapeDtypeStruct((NC, NPAD, 128), jnp.float32),
        mesh=_mesh(),
        scratch_types=[
            pltpu.VMEM((C2, K), jnp.int32),
            pltpu.VMEM((K, 128), jnp.float32),
            pltpu.VMEM((ZROWS, 128), jnp.float32),
            pltpu.VMEM_SHARED((NPAD, 128), jnp.float32),
            pltpu.SemaphoreType.DMA,
        ],
    )


def _pipelined_edges(tbl, rows_hbm, ring, col_v, bufs, agg_sh,
                     gsem, ssem, isem, chunks):
    assert chunks % 8 == 0

    for q in range(4):
        pltpu.async_copy(rows_hbm.at[q], ring.at[q], isem[q])

    @pl.loop(0, chunks // 8)
    def _outer(g):
        for jj in range(8):
            j = g * 8 + jj
            b, q, bp = jj % 2, jj % 4, (jj - 1) % 2

            def _free_and_refill(j=j, b=b, q=q):
                pltpu.make_async_copy(
                    bufs[b], agg_sh.at[col_v.at[j]], ssem[b]).wait()

                @pl.when(j < chunks - 2)
                def _():
                    pltpu.async_copy(rows_hbm.at[j + 2],
                                     ring.at[(q + 2) % 4], isem[(q + 2) % 4])

            if jj < 2:
                @pl.when(g > 0)
                def _(f=_free_and_refill):
                    f()
            else:
                _free_and_refill()

            pltpu.make_async_copy(rows_hbm.at[j], ring.at[q], isem[q]).wait()
            pltpu.async_copy(tbl.at[ring.at[q]], bufs[b], gsem[b])

            def _scatter_prev(j=j, bp=bp):
                pltpu.make_async_copy(
                    tbl.at[ring.at[0]], bufs[bp], gsem[bp]).wait()
                pltpu.async_copy(bufs[bp], agg_sh.at[col_v.at[j - 1]],
                                 ssem[bp], add=True)

            if jj == 0:
                @pl.when(g > 0)
                def _(f=_scatter_prev):
                    f()
            else:
                _scatter_prev()

    last = chunks - 1
    pltpu.make_async_copy(tbl.at[ring.at[0]], bufs[last % 2],
                          gsem[last % 2]).wait()
    pltpu.async_copy(bufs[last % 2], agg_sh.at[col_v.at[last]],
                     ssem[last % 2], add=True)
    for b in range(2):
        pltpu.make_async_copy(bufs[b], agg_sh.at[col_v.at[0]],
                              ssem[b]).wait()


def _make_agg(W):
    def body(tbl, rows2, col3, out, col_v, ring, b0, b1, zbuf,
             agg_sh, g0, g1, s0, s1, i0, i1, i2, i3):
        c = lax.axis_index("c")
        s = lax.axis_index("s")
        r0 = s * ROWS_PER_TILE
        _zero_spmem_slice(zbuf, agg_sh, r0)
        pltpu.sync_copy(col3.at[s], col_v)
        plsc.subcore_barrier()
        _pipelined_edges(tbl, rows2.at[c, s], ring, col_v, [b0, b1], agg_sh,
                         [g0, g1], [s0, s1], [i0, i1, i2, i3], C1)
        plsc.subcore_barrier()
        pltpu.sync_copy(agg_sh.at[pl.ds(r0, ROWS_PER_TILE)],
                        out.at[c, pl.ds(r0, ROWS_PER_TILE)])

    return pl.kernel(
        body,
        out_type=jax.ShapeDtypeStruct((NC, NPAD, W), jnp.float32),
        mesh=_mesh(),
        scratch_types=[
            pltpu.VMEM((C1, K), jnp.int32),
            pltpu.VMEM((4, K), jnp.int32),
            pltpu.VMEM((K, W), jnp.float32),
            pltpu.VMEM((K, W), jnp.float32),
            pltpu.VMEM((ZROWS, 128), jnp.float32),
            pltpu.VMEM_SHARED((NPAD, W), jnp.float32),
            pltpu.SemaphoreType.DMA,
            pltpu.SemaphoreType.DMA,
            pltpu.SemaphoreType.DMA,
            pltpu.SemaphoreType.DMA,
            pltpu.SemaphoreType.DMA,
            pltpu.SemaphoreType.DMA,
            pltpu.SemaphoreType.DMA,
            pltpu.SemaphoreType.DMA,
        ],
    )


def _make_agg2():
    def body(tbl, row4, col4, out, col_v, ring, b0, b1, zbuf,
             agg_sh, g0, g1, s0, s1, i0, i1, i2, i3):
        c = lax.axis_index("c")
        s = lax.axis_index("s")
        r0 = s * ROWS_PER_TILE
        _zero_spmem_slice(zbuf, agg_sh, r0)
        pltpu.sync_copy(col4.at[c, s], col_v)
        plsc.subcore_barrier()
        _pipelined_edges(tbl, row4.at[c, s], ring, col_v, [b0, b1], agg_sh,
                         [g0, g1], [s0, s1], [i0, i1, i2, i3], C2)
        plsc.subcore_barrier()
        pltpu.sync_copy(agg_sh.at[pl.ds(r0, ROWS_PER_TILE)],
                        out.at[c, pl.ds(r0, ROWS_PER_TILE)])

    return pl.kernel(
        body,
        out_type=jax.ShapeDtypeStruct((NC, NPAD, 128), jnp.float32),
        mesh=_mesh(),
        scratch_types=[
            pltpu.VMEM((C2, K), jnp.int32),
            pltpu.VMEM((4, K), jnp.int32),
            pltpu.VMEM((K, 128), jnp.float32),
            pltpu.VMEM((K, 128), jnp.float32),
            pltpu.VMEM((ZROWS, 128), jnp.float32),
            pltpu.VMEM_SHARED((NPAD, 128), jnp.float32),
            pltpu.SemaphoreType.DMA,
            pltpu.SemaphoreType.DMA,
            pltpu.SemaphoreType.DMA,
            pltpu.SemaphoreType.DMA,
            pltpu.SemaphoreType.DMA,
            pltpu.SemaphoreType.DMA,
            pltpu.SemaphoreType.DMA,
            pltpu.SemaphoreType.DMA,
        ],
    )


def _tc0_body(x_ref, w1_ref, v1_ref, b1_ref, h_ref, xv_ref):
    h_ref[...] = jnp.dot(x_ref[...], w1_ref[...],
                         preferred_element_type=jnp.float32)
    xv_ref[...] = (jnp.dot(x_ref[...], v1_ref[...],
                           preferred_element_type=jnp.float32)
                   + b1_ref[...])


def _tc1_body(h_ref, deg_ref, hs_ref, dis_ref):
    deg = deg_ref[0, :, 0] + deg_ref[1, :, 0]
    dis = jnp.where(deg > 0, lax.rsqrt(deg), 0.0)
    dis_ref[0, 0] = dis
    hs = h_ref[...] * dis[:, None]
    hs_ref[0] = hs[:, :128]
    hs_ref[1] = hs[:, 128:]


def _tc2_body(agg_ref, dis_ref, xv_ref, w2_ref, v2_ref, b2_ref,
              hs2_ref, xv2_ref):
    dis = dis_ref[0, 0]
    agg = jnp.concatenate([agg_ref[0], agg_ref[1]], axis=1)
    out1 = jnp.maximum(agg * dis[:, None] + xv_ref[...], 0.0)
    h2 = jnp.dot(out1, w2_ref[...], preferred_element_type=jnp.float32)
    hs2 = h2 * dis[:, None]
    hs2_ref[...] = jnp.concatenate(
        [hs2, jnp.zeros((RB, 128 - D_OUT2), jnp.float32)], axis=1)
    xv2_ref[...] = (jnp.dot(out1, v2_ref[...],
                            preferred_element_type=jnp.float32)
                    + b2_ref[...])


def _tc3_body(agg_ref, dis_ref, xv2_ref, batch_ref, fcw_ref, fcb_ref,
              out_ref, acc_sum, acc_cnt):
    i = pl.program_id(0)

    @pl.when(i == 0)
    def _():
        acc_sum[...] = jnp.zeros_like(acc_sum)
        acc_cnt[...] = jnp.zeros_like(acc_cnt)

    dis = dis_ref[0, 0]
    agg = agg_ref[0, :, :D_OUT2] + agg_ref[1, :, :D_OUT2]
    out2 = jnp.maximum(agg * dis[:, None] + xv2_ref[...], 0.0)
    b = batch_ref[0, 0]
    onehot = (b[:, None] == lax.broadcasted_iota(jnp.int32, (RB, N_GRAPHS), 1)
              ).astype(jnp.float32)
    acc_sum[...] += lax.dot_general(onehot, out2, (((0,), (0,)), ((), ())),
                                    preferred_element_type=jnp.float32)
    acc_cnt[...] += jnp.sum(onehot, axis=0, keepdims=True)

    @pl.when(i == GRID - 1)
    def _():
        pooled = acc_sum[...] / jnp.maximum(acc_cnt[...], 1.0).T
        out_ref[...] = (jnp.dot(pooled, fcw_ref[...],
                                preferred_element_type=jnp.float32)
                        + fcb_ref[...])


def _tc0(x, W1, V1, b1):
    return pl.pallas_call(
        _tc0_body,
        grid=(GRID,),
        in_specs=[
            pl.BlockSpec((RB, D_IN), lambda i: (i, 0)),
            pl.BlockSpec((D_IN, D_HID), lambda i: (0, 0)),
            pl.BlockSpec((D_IN, D_HID), lambda i: (0, 0)),
            pl.BlockSpec((1, D_HID), lambda i: (0, 0)),
        ],
        out_specs=[
            pl.BlockSpec((RB, D_HID), lambda i: (i, 0)),
            pl.BlockSpec((RB, D_HID), lambda i: (i, 0)),
        ],
        out_shape=[
            jax.ShapeDtypeStruct((N, D_HID), jnp.float32),
            jax.ShapeDtypeStruct((N, D_HID), jnp.float32),
        ],
    )(x, W1, V1, b1)


def _tc1(h1, degS):
    return pl.pallas_call(
        _tc1_body,
        grid=(GRID,),
        in_specs=[
            pl.BlockSpec((RB, D_HID), lambda i: (i, 0)),
            pl.BlockSpec((NC, RB, 128), lambda i: (0, i, 0)),
        ],
        out_specs=[
            pl.BlockSpec((NC, RB, 128), lambda i: (0, i, 0)),
            pl.BlockSpec((1, 1, RB), lambda i: (i, 0, 0)),
        ],
        out_shape=[
            jax.ShapeDtypeStruct((NC, N, 128), jnp.float32),
            jax.ShapeDtypeStruct((GRID, 1, RB), jnp.float32),
        ],
    )(h1, degS)


def _tc2(agg1, dis3, xv1, W2, V2, b2):
    return pl.pallas_call(
        _tc2_body,
        grid=(GRID,),
        in_specs=[
            pl.BlockSpec((NC, RB, 128), lambda i: (0, i, 0)),
            pl.BlockSpec((1, 1, RB), lambda i: (i, 0, 0)),
            pl.BlockSpec((RB, D_HID), lambda i: (i, 0)),
            pl.BlockSpec((D_HID, D_OUT2), lambda i: (0, 0)),
            pl.BlockSpec((D_HID, D_OUT2), lambda i: (0, 0)),
            pl.BlockSpec((1, D_OUT2), lambda i: (0, 0)),
        ],
        out_specs=[
            pl.BlockSpec((RB, 128), lambda i: (i, 0)),
            pl.BlockSpec((RB, D_OUT2), lambda i: (i, 0)),
        ],
        out_shape=[
            jax.ShapeDtypeStruct((N, 128), jnp.float32),
            jax.ShapeDtypeStruct((N, D_OUT2), jnp.float32),
        ],
    )(agg1, dis3, xv1, W2, V2, b2)


def _tc3(agg2, dis3, xv2, batch2, fc_w, fc_b):
    return pl.pallas_call(
        _tc3_body,
        grid=(GRID,),
        in_specs=[
            pl.BlockSpec((NC, RB, 128), lambda i: (0, i, 0)),
            pl.BlockSpec((1, 1, RB), lambda i: (i, 0, 0)),
            pl.BlockSpec((RB, D_OUT2), lambda i: (i, 0)),
            pl.BlockSpec((1, 1, RB), lambda i: (i, 0, 0)),
            pl.BlockSpec((D_OUT2, 10), lambda i: (0, 0)),
            pl.BlockSpec((1, 10), lambda i: (0, 0)),
        ],
        out_specs=pl.BlockSpec((N_GRAPHS, 10), lambda i: (0, 0)),
        out_shape=jax.ShapeDtypeStruct((N_GRAPHS, 10), jnp.float32),
        scratch_shapes=[
            pltpu.VMEM((N_GRAPHS, D_OUT2), jnp.float32),
            pltpu.VMEM((1, N_GRAPHS), jnp.float32),
        ],
    )(agg2, dis3, xv2, batch2, fc_w, fc_b)


def kernel(x, edge_index, batch, W1, V1, b1, W2, V2, b2, fc_w, fc_b):
    ei = edge_index.astype(jnp.int32)
    row, col = ei[0], ei[1]

    pad = E_PAD - E
    spread = jnp.arange(pad, dtype=jnp.int32) % 64
    rowp = jnp.concatenate([row, spread])
    colp = jnp.concatenate([col, 10016 + spread])
    rows2 = jnp.stack([rowp, rowp + N]).reshape(NC, NS, C1, K)
    col3 = colp.reshape(NS, C1, K)
    col4 = colp.reshape(NC, NS, C2, K)
    row4 = rowp.reshape(NC, NS, C2, K)
    batch2 = batch.astype(jnp.int32).reshape(GRID, 1, RB)

    ones128 = jnp.ones((K, 128), jnp.float32)

    degS = _make_deg()(col4, ones128)
    h1, xv1 = _tc0(x, W1, V1, b1.reshape(1, D_HID))
    hs1, dis3 = _tc1(h1, degS)
    agg1 = _make_agg(128)(hs1.reshape(NC * N, 128), rows2, col3)

    hs2, xv2 = _tc2(agg1, dis3, xv1, W2, V2, b2.reshape(1, D_OUT2))
    agg2 = _make_agg2()(hs2, row4, col4)

    return _tc3(agg2, dis3, xv2, batch2, fc_w, fc_b.reshape(1, 10))
